# Initial kernel scaffold; baseline (speedup 1.0000x reference)
import jax
import jax.numpy as jnp
from jax.experimental import pallas as pl


def kernel(x, params):
    raise NotImplementedError("write your pallas kernel here")


# speedup vs baseline: 14.0795x; 14.0795x over previous
import jax
import jax.numpy as jnp
from jax.experimental import pallas as pl

H = 128
N = 10


def _lin(h, p):
    return (
        jnp.dot(
            h,
            p["W"],
            preferred_element_type=jnp.float32,
        )
        + p["b"]
    )


def _bn(h, p):
    m = jnp.mean(h, axis=0, keepdims=True)
    v = jnp.mean((h - m) ** 2, axis=0, keepdims=True)
    return (h - m) / jnp.sqrt(v + 1e-5) * p["g"] + p["b"]


def _lin_k1(x, p):
    return x * p["W"] + p["b"]


def _genconv(p, x, ea_col, mask_col, has_lin):
    if has_lin:
        if x.shape[1] == 1:
            h_src = _lin_k1(x, p["lin_src"])
            x_dst = _lin_k1(x, p["lin_dst"])
        else:
            h_src = _lin(x, p["lin_src"])
            x_dst = _lin(x, p["lin_dst"])
    else:
        h_src = x
        x_dst = x
    e_lin = _lin_k1(ea_col, p["lin_edge"])
    blocks = []
    masks = []
    mx = jnp.full((N, x_dst.shape[1]), -jnp.inf, jnp.float32)
    for i in range(N):
        blk = jax.nn.relu(h_src[i : i + 1, :] + e_lin[i * N : (i + 1) * N, :]) + 1e-7
        m = mask_col[i * N : (i + 1) * N, :]
        blocks.append(blk)
        masks.append(m)
        mx = jnp.maximum(mx, jnp.where(m, blk, -jnp.inf))
    num = jnp.zeros_like(mx)
    den = jnp.zeros_like(mx)
    for blk, m in zip(blocks, masks):
        a = jnp.where(m, jnp.exp(blk - mx), 0.0)
        den = den + a
        num = num + a * blk
    out = num / den + x_dst
    h = _lin(out, p["mlp1"])
    h = jax.nn.relu(_bn(h, p["mlp_bn"]))
    return _lin(h, p["mlp2"])


def _branch(p, feat, ea_col, mask_col):
    h = _genconv(p["conv1"], feat, ea_col, mask_col, True)
    h = jax.nn.relu(_bn(h, p["bn1"]))
    h = _genconv(p["conv2"], h, ea_col, mask_col, False)
    h = jax.nn.relu(_bn(h, p["bn2"]))
    for lp in p["lins"][:-1]:
        h = jax.nn.relu(_lin(h, lp))
    h = _lin(h, p["lins"][-1])
    s = _lin(h, p["att"])
    s = jnp.exp(s - jnp.max(s, axis=0, keepdims=True))
    s = s / jnp.sum(s, axis=0, keepdims=True)
    return jnp.sum(s * h, axis=0, keepdims=True)


def _mlp(ps, h):
    for lp in ps[:-1]:
        h = jax.nn.relu(_lin(h, lp))
    return _lin(h, ps[-1])


def _forward(topo, t_col, f_col, routing, params):
    t_mask = t_col != 0.0
    f_mask = f_col != 0.0
    t_ea = t_col / jnp.sum(t_col)
    f_ea = f_col / jnp.sum(f_col)
    topo_feat = jnp.sum(topo, axis=1, keepdims=True) / jnp.sum(topo)
    traf_feat = jnp.eye(N, dtype=jnp.float32)
    out_t = _branch(params["topology"], topo_feat, t_ea, t_mask)
    out_f = _branch(params["traffic"], traf_feat, f_ea, f_mask)
    out_r = _mlp(params["routing"], routing)
    cat = jnp.concatenate([out_t, out_f, out_r], axis=1)
    return _mlp(params["joint"], cat)


def _fused_body(treedef, topo_ref, t_col_ref, f_col_ref, routing_ref, *refs):
    out_ref = refs[-1]
    leaves = [r[...] for r in refs[:-1]]
    params = jax.tree_util.tree_unflatten(treedef, leaves)
    out_ref[...] = _forward(
        topo_ref[...], t_col_ref[...], f_col_ref[...], routing_ref[...], params
    )


def kernel(x, params):
    leaves, treedef = jax.tree_util.tree_flatten(params)
    leaves2 = [l.reshape(1, -1) if l.ndim == 1 else l for l in leaves]
    topo = x[0]
    t_col = x[0].reshape(N * N, 1)
    f_col = x[1].reshape(N * N, 1)
    routing = x[2].reshape(1, N * N)
    body = lambda *refs: _fused_body(treedef, *refs)
    out = pl.pallas_call(
        body,
        out_shape=jax.ShapeDtypeStruct((1, 64), jnp.float32),
    )(topo, t_col, f_col, routing, *leaves2)
    return out.reshape(64)


# speedup vs baseline: 15.7883x; 1.1214x over previous
import jax
import jax.numpy as jnp
from jax.experimental import pallas as pl

H = 128
N = 10


def _lin(h, p):
    return jnp.dot(h, p["W"], preferred_element_type=jnp.float32) + p["b"]


def _lin_k1(x, p):
    return x * p["W"] + p["b"]


def _bn(h, p):
    m = jnp.mean(h, axis=0, keepdims=True)
    v = jnp.mean((h - m) ** 2, axis=0, keepdims=True)
    return (h - m) / jnp.sqrt(v + 1e-5) * p["g"] + p["b"]


def _genconv(p, x, ea, mask, has_lin):
    if has_lin:
        if x.shape[1] == 1:
            h_src = _lin_k1(x, p["lin_src"])
            x_dst = _lin_k1(x, p["lin_dst"])
        else:
            h_src = _lin(x, p["lin_src"])
            x_dst = _lin(x, p["lin_dst"])
    else:
        h_src = x
        x_dst = x
    we = p["lin_edge"]["W"]
    be = p["lin_edge"]["b"]
    out_rows = []
    for j in range(N):
        ea_j = ea[:, j : j + 1]
        m_j = mask[:, j : j + 1]
        blk = jax.nn.relu(h_src + (ea_j * we + be)) + 1e-7
        blkm = jnp.where(m_j, blk, -jnp.inf)
        mx = blkm[0:1, :]
        for i in range(1, N):
            mx = jnp.maximum(mx, blkm[i : i + 1, :])
        a = jnp.where(m_j, jnp.exp(blk - mx), 0.0)
        am = a * blk
        den = a[0:1, :]
        num = am[0:1, :]
        for i in range(1, N):
            den = den + a[i : i + 1, :]
            num = num + am[i : i + 1, :]
        out_rows.append(num / den)
    out = jnp.concatenate(out_rows, axis=0) + x_dst
    h = _lin(out, p["mlp1"])
    h = jax.nn.relu(_bn(h, p["mlp_bn"]))
    return _lin(h, p["mlp2"])


def _branch(p, feat, ea, mask):
    h = _genconv(p["conv1"], feat, ea, mask, True)
    h = jax.nn.relu(_bn(h, p["bn1"]))
    h = _genconv(p["conv2"], h, ea, mask, False)
    h = jax.nn.relu(_bn(h, p["bn2"]))
    for lp in p["lins"][:-1]:
        h = jax.nn.relu(_lin(h, lp))
    h = _lin(h, p["lins"][-1])
    s = _lin(h, p["att"])
    s = jnp.exp(s - jnp.max(s, axis=0, keepdims=True))
    s = s / jnp.sum(s, axis=0, keepdims=True)
    return jnp.sum(s * h, axis=0, keepdims=True)


def _mlp(ps, h):
    for lp in ps[:-1]:
        h = jax.nn.relu(_lin(h, lp))
    return _lin(h, ps[-1])


def _forward(x, routing, params):
    topo = x[0]
    traf = x[1]
    t_mask = topo != 0.0
    f_mask = traf != 0.0
    t_ea = topo / jnp.sum(topo)
    f_ea = traf / jnp.sum(traf)
    topo_feat = jnp.sum(topo, axis=1, keepdims=True) / jnp.sum(topo)
    traf_feat = jnp.eye(N, dtype=jnp.float32)
    out_t = _branch(params["topology"], topo_feat, t_ea, t_mask)
    out_f = _branch(params["traffic"], traf_feat, f_ea, f_mask)
    out_r = _mlp(params["routing"], routing)
    cat = jnp.concatenate([out_t, out_f, out_r], axis=1)
    return _mlp(params["joint"], cat)


def _fused_body(treedef, x_ref, routing_ref, *refs):
    out_ref = refs[-1]
    leaves = [r[...] for r in refs[:-1]]
    params = jax.tree_util.tree_unflatten(treedef, leaves)
    out_ref[...] = _forward(x_ref[...], routing_ref[...], params)


def kernel(x, params):
    leaves, treedef = jax.tree_util.tree_flatten(params)
    leaves2 = [l.reshape(1, -1) if l.ndim == 1 else l for l in leaves]
    routing = x[2].reshape(1, N * N)
    body = lambda *refs: _fused_body(treedef, *refs)
    out = pl.pallas_call(
        body,
        out_shape=jax.ShapeDtypeStruct((1, 64), jnp.float32),
    )(x, routing, *leaves2)
    return out.reshape(64)
